# Initial kernel scaffold; baseline (speedup 1.0000x reference)
import jax
import jax.numpy as jnp
from jax.experimental import pallas as pl


def kernel(x, edge_index, cp_w1, cp_b1, cp_w2, cp_b2, w1, b1, w2, b2, w3, b3, bn1_g, bn1_b, bn1_rm, bn1_rv, bn2_g, bn2_b, bn2_rm, bn2_rv, bn3_g, bn3_b, bn3_rm, bn3_rv, ci_w, ci_b, cls_w, cls_b):
    raise NotImplementedError("write your pallas kernel here")


# speedup vs baseline: 11.5195x; 11.5195x over previous
import functools

import jax
import jax.numpy as jnp
from jax import lax
from jax.experimental import pallas as pl
from jax.experimental.pallas import tpu as pltpu
from jax.experimental.pallas import tpu_sc as plsc

N = 50000
E = 800000
CHUNK = 128
ECH = E // CHUNK
NC = 2
NT = 16
RPT = N // NT
ZROWS = 125
B = 2000
NB = N // B


@functools.cache
def _sc_pass(F, gather, split_edges, idx_stride):
    chunks_per_core = ECH // NC if split_edges else ECH
    q, r = divmod(chunks_per_core, NT)

    def body(src_ref, dst_ref, tbl_ref, out_ref, sbuf, dbuf, rows, zbuf,
             acc, gsem):
        c = lax.axis_index("c")
        s = lax.axis_index("s")

        def zrow(i, carry):
            for t in range(F // 16):
                zbuf[i, pl.ds(t * 16, 16)] = jnp.zeros((16,), jnp.float32)
            return carry
        lax.fori_loop(0, ZROWS, zrow, 0)
        if not gather:
            def orow(i, carry):
                for t in range(F // 16):
                    rows[i, pl.ds(t * 16, 16)] = jnp.ones((16,), jnp.float32)
                return carry
            lax.fori_loop(0, CHUNK, orow, 0)

        def zcp(k, carry):
            pltpu.sync_copy(zbuf, acc.at[pl.ds(s * RPT + k * ZROWS, ZROWS), :])
            return carry
        lax.fori_loop(0, RPT // ZROWS, zcp, 0)
        plsc.subcore_barrier()

        cnt = q + jnp.where(s < r, 1, 0)
        start = s * q + jnp.minimum(s, r)
        if split_edges:
            start = start + c * chunks_per_core

        def step(i, carry):
            pltpu.sync_copy(dst_ref.at[pl.ds(i, 1), :], dbuf)
            if gather:
                pltpu.sync_copy(src_ref.at[pl.ds(i, 1), :], sbuf)
                if idx_stride:
                    off = c * idx_stride
                    for t in range(CHUNK // 16):
                        sl = pl.ds(t * 16, 16)
                        sbuf[0, sl] = sbuf[0, sl] + off
                pltpu.async_copy(tbl_ref.at[sbuf.at[0]], rows, gsem).wait()
            pltpu.sync_copy(rows, acc.at[dbuf.at[0]], add=True)
            return carry
        lax.fori_loop(start, start + cnt, step, 0)
        plsc.subcore_barrier()

        pltpu.sync_copy(acc.at[pl.ds(s * RPT, RPT), :],
                        out_ref.at[pl.ds(c * N + s * RPT, RPT), :])

    return pl.kernel(
        body,
        out_type=jax.ShapeDtypeStruct((NC * N, F), jnp.float32),
        mesh=plsc.VectorSubcoreMesh(core_axis_name="c", subcore_axis_name="s",
                                    num_cores=NC, num_subcores=NT),
        compiler_params=pltpu.CompilerParams(use_tc_tiling_on_sc=False),
        scratch_types=[
            pltpu.VMEM((1, CHUNK), jnp.int32),
            pltpu.VMEM((1, CHUNK), jnp.int32),
            pltpu.VMEM((CHUNK, F), jnp.float32),
            pltpu.VMEM((ZROWS, F), jnp.float32),
            pltpu.VMEM_SHARED((N, F), jnp.float32),
            pltpu.SemaphoreType.DMA,
        ],
    )


def _deg_pass(s, d, t):
    return _sc_pass(16, gather=False, split_edges=True, idx_stride=0)(s, d, t)


def _agg1_pass(s, d, t):
    return _sc_pass(16, gather=True, split_edges=True, idx_stride=0)(s, d, t)


def _agg2_pass(s, d, t):
    return _sc_pass(32, gather=True, split_edges=False, idx_stride=N)(s, d, t)


def _agg3_pass(s, d, t):
    return _sc_pass(16, gather=True, split_edges=False, idx_stride=N)(s, d, t)


def _bn_st(g, b, rm, rv):
    s = g * lax.rsqrt(rv + 1e-5)
    return s, b - rm * s


def _tc1_body(d0, d1, x16, dis_o, y1_o):
    disv = lax.rsqrt(d0[...] + d1[...] + 1.0)
    dis_o[...] = disv[:, :1]
    y1_o[...] = disv * x16[...]


def _tc2_body(p0, p1, x16, dis, w1, b1, g, bb, rm, rv, w2h, y2_o):
    dis_v = dis[...]
    aggx = dis_v * (p0[...] + p1[...]) + (dis_v * dis_v) * x16[...]
    z = jnp.dot(aggx, w1[...], preferred_element_type=jnp.float32) + b1[...]
    s, t = _bn_st(g[...], bb[...], rm[...], rv[...])
    h1 = jnp.maximum(z * s + t, 0.0)
    y2_o[0] = dis_v * jnp.dot(h1, w2h[0], preferred_element_type=jnp.float32)


def _tc3_body(a2a, a2b, y2, dis, b2, g, bb, rm, rv, w3h, y3_o):
    dis_v = dis[...]
    conv = jnp.concatenate(
        [dis_v * (a2a[...] + y2[0]), dis_v * (a2b[...] + y2[1])], axis=1)
    conv = conv + b2[...]
    s, t = _bn_st(g[...], bb[...], rm[...], rv[...])
    h2 = jnp.maximum(conv * s + t, 0.0)
    y3_o[0] = dis_v * jnp.dot(h2, w3h[0], preferred_element_type=jnp.float32)


def _tc4_body(a3a, a3b, y3, dis, x16, b3, g, bb, rm, rv,
              cp_w1, cp_b1, cp_w2, cp_b2, ci_w, ci_b, cls_w, cls_b, out_o):
    dis_v = dis[...]
    conv = jnp.concatenate(
        [dis_v * (a3a[...] + y3[0]), dis_v * (a3b[...] + y3[1])], axis=1)
    conv = conv + b3[...]
    s, t = _bn_st(g[...], bb[...], rm[...], rv[...])
    h3 = jnp.maximum(conv * s + t, 0.0)
    ch = x16[:, 8:9]
    pc = jnp.maximum(ch * cp_w1[...] + cp_b1[...], 0.0)
    pc = jnp.dot(pc, cp_w2[...], preferred_element_type=jnp.float32) + cp_b2[...]
    ci = ci_w[...]
    hh = (jnp.dot(h3, ci[:32, :], preferred_element_type=jnp.float32)
          + jnp.dot(pc, ci[32:, :], preferred_element_type=jnp.float32)
          + ci_b[...])
    hh = jnp.maximum(hh, 0.0)
    logit = jnp.dot(hh, cls_w[...], preferred_element_type=jnp.float32) + cls_b[...]
    out_o[...] = jax.nn.sigmoid(logit)


def _row_spec(fdim, offset_blocks=0):
    return pl.BlockSpec((B, fdim), lambda i, *_: (i + offset_blocks, 0))


def _full_spec(shape):
    return pl.BlockSpec(shape, lambda i, *_: tuple(0 for _ in shape))


def _build_tc(interpret=False):
    tc1 = pl.pallas_call(
        _tc1_body,
        grid=(NB,),
        in_specs=[_row_spec(16), _row_spec(16, NB), _row_spec(16)],
        out_specs=[_row_spec(1), _row_spec(16)],
        out_shape=[jax.ShapeDtypeStruct((N, 1), jnp.float32),
                   jax.ShapeDtypeStruct((N, 16), jnp.float32)],
        interpret=interpret,
    )
    tc2 = pl.pallas_call(
        _tc2_body,
        grid=(NB, 2),
        in_specs=[_row_spec(16), _row_spec(16, NB), _row_spec(16), _row_spec(1),
                  _full_spec((16, 128)), _full_spec((1, 128)),
                  _full_spec((1, 128)), _full_spec((1, 128)),
                  _full_spec((1, 128)), _full_spec((1, 128)),
                  pl.BlockSpec((1, 128, 32), lambda i, j: (j, 0, 0))],
        out_specs=pl.BlockSpec((1, B, 32), lambda i, j: (j, i, 0)),
        out_shape=jax.ShapeDtypeStruct((2, N, 32), jnp.float32),
        interpret=interpret,
    )
    tc3 = pl.pallas_call(
        _tc3_body,
        grid=(NB, 2),
        in_specs=[_row_spec(32), _row_spec(32, NB),
                  pl.BlockSpec((2, B, 32), lambda i, j: (0, i, 0)),
                  _row_spec(1), _full_spec((1, 64)),
                  _full_spec((1, 64)), _full_spec((1, 64)),
                  _full_spec((1, 64)), _full_spec((1, 64)),
                  pl.BlockSpec((1, 64, 16), lambda i, j: (j, 0, 0))],
        out_specs=pl.BlockSpec((1, B, 16), lambda i, j: (j, i, 0)),
        out_shape=jax.ShapeDtypeStruct((2, N, 16), jnp.float32),
        interpret=interpret,
    )
    tc4 = pl.pallas_call(
        _tc4_body,
        grid=(NB,),
        in_specs=[_row_spec(16), _row_spec(16, NB),
                  pl.BlockSpec((2, B, 16), lambda i: (0, i, 0)),
                  _row_spec(1), _row_spec(16), _full_spec((1, 32)),
                  _full_spec((1, 32)), _full_spec((1, 32)),
                  _full_spec((1, 32)), _full_spec((1, 32)),
                  _full_spec((1, 32)), _full_spec((1, 32)),
                  _full_spec((32, 16)), _full_spec((1, 16)),
                  _full_spec((48, 32)), _full_spec((1, 32)),
                  _full_spec((32, 1)), _full_spec((1, 1))],
        out_specs=_row_spec(1),
        out_shape=jax.ShapeDtypeStruct((N, 1), jnp.float32),
        interpret=interpret,
    )
    return tc1, tc2, tc3, tc4


_tc1, _tc2, _tc3, _tc4 = _build_tc()


def kernel(x, edge_index, cp_w1, cp_b1, cp_w2, cp_b2, w1, b1, w2, b2, w3, b3,
           bn1_g, bn1_b, bn1_rm, bn1_rv, bn2_g, bn2_b, bn2_rm, bn2_rv,
           bn3_g, bn3_b, bn3_rm, bn3_rv, ci_w, ci_b, cls_w, cls_b):
    src2d = edge_index[0].reshape(ECH, CHUNK)
    dst2d = edge_index[1].reshape(ECH, CHUNK)
    x16 = jnp.pad(x, ((0, 0), (0, 6)))
    w1p = jnp.pad(w1, ((0, 6), (0, 0)))
    r2 = lambda v: v.reshape(1, -1)
    _colsplit = lambda w: w.reshape(w.shape[0], 2, -1).transpose(1, 0, 2)

    deg = _deg_pass(src2d, dst2d, x16)
    dis, y1 = _tc1(deg, deg, x16)
    l1 = _agg1_pass(src2d, dst2d, y1)
    y2 = _tc2(l1, l1, x16, dis, w1p, r2(b1), r2(bn1_g), r2(bn1_b),
              r2(bn1_rm), r2(bn1_rv), _colsplit(w2))
    l2 = _agg2_pass(src2d, dst2d, y2.reshape(NC * N, 32))
    y3 = _tc3(l2, l2, y2, dis, r2(b2), r2(bn2_g), r2(bn2_b),
              r2(bn2_rm), r2(bn2_rv), _colsplit(w3))
    l3 = _agg3_pass(src2d, dst2d, y3.reshape(NC * N, 16))
    out = _tc4(l3, l3, y3, dis, x16, r2(b3), r2(bn3_g), r2(bn3_b),
               r2(bn3_rm), r2(bn3_rv), cp_w1, r2(cp_b1), cp_w2, r2(cp_b2),
               ci_w, r2(ci_b), cls_w, r2(cls_b))
    return out


# speedup vs baseline: 27.1563x; 2.3574x over previous
import functools

import jax
import jax.numpy as jnp
from jax import lax
from jax.experimental import pallas as pl
from jax.experimental.pallas import tpu as pltpu
from jax.experimental.pallas import tpu_sc as plsc

N = 50000
E = 800000
CHUNK = 128
ECH = E // CHUNK
NC = 2
NT = 16
RPT = N // NT
ZROWS = 125
B = 2000
NB = N // B


@functools.cache
def _sc_pass(F, gather, split_edges, idx_stride):
    U = 4 if F > 16 else 8
    UH = U // 2
    cpc = ECH // NC if split_edges else ECH
    groups, leftover = divmod(cpc, U)
    qg, rg = divmod(groups, NT)

    def body(ei_ref, tbl_ref, out_ref, ibuf, rows, zbuf, acc, *sems):
        gsem, ssem = sems[:U], sems[U:]
        c = lax.axis_index("c")
        s = lax.axis_index("s")
        off = c * idx_stride

        def zrow(i, carry):
            for t in range(F // 16):
                zbuf[i, pl.ds(t * 16, 16)] = jnp.zeros((16,), jnp.float32)
            return carry
        lax.fori_loop(0, ZROWS, zrow, 0)
        if not gather:
            def orow(i, carry):
                for t in range(F // 16):
                    rows[0, i, pl.ds(t * 16, 16)] = jnp.ones((16,), jnp.float32)
                return carry
            lax.fori_loop(0, CHUNK, orow, 0)

        def zcp(k, carry):
            pltpu.sync_copy(zbuf, acc.at[pl.ds(s * RPT + k * ZROWS, ZROWS), :])
            return carry
        lax.fori_loop(0, RPT // ZROWS, zcp, 0)
        plsc.subcore_barrier()

        base_c = c * cpc if split_edges else 0
        gcnt = qg + jnp.where(s < rg, 1, 0)
        gstart = base_c + (s * qg + jnp.minimum(s, rg)) * U

        def sidx(u):
            return ibuf.at[u, pl.ds(0, CHUNK)]

        def didx(u):
            return ibuf.at[u, pl.ds(CHUNK, CHUNK)]

        def scat_desc(u):
            return pltpu.make_async_copy(
                rows.at[u], acc.at[didx(u)], ssem[u])

        def do_group(chunk0, drain):
            for h in range(U // UH):
                u0 = h * UH
                if drain:
                    for u in range(u0, u0 + UH):
                        scat_desc(u).wait()
                pltpu.sync_copy(ei_ref.at[pl.ds(chunk0 + u0, UH), :],
                                ibuf.at[pl.ds(u0, UH), :])
                if gather and idx_stride:
                    for u in range(u0, u0 + UH):
                        for t in range(CHUNK // 16):
                            sl = pl.ds(t * 16, 16)
                            ibuf[u, sl] = ibuf[u, sl] + off
                if gather:
                    for u in range(u0, u0 + UH):
                        pltpu.async_copy(tbl_ref.at[sidx(u)], rows.at[u],
                                         gsem[u])
            for u in range(U):
                if gather:
                    pltpu.make_async_copy(tbl_ref.at[sidx(u)], rows.at[u],
                                          gsem[u]).wait()
                    src = rows.at[u]
                else:
                    src = rows.at[0]
                pltpu.async_copy(src, acc.at[didx(u)], ssem[u], add=True)

        do_group(gstart, drain=False)

        def grp(i, carry):
            do_group(gstart + i * U, drain=True)
            return carry
        lax.fori_loop(1, gcnt, grp, 0)
        for u in range(U):
            if gather:
                pltpu.make_async_copy(rows.at[u], acc.at[didx(u)],
                                      ssem[u]).wait()
            else:
                pltpu.make_async_copy(rows.at[0], acc.at[didx(u)],
                                      ssem[u]).wait()

        if leftover:
            @pl.when(s == NT - 1)
            def _tail():
                for t in range(leftover):
                    ch = base_c + groups * U + t
                    pltpu.sync_copy(ei_ref.at[pl.ds(ch, 1), :],
                                    ibuf.at[pl.ds(0, 1), :])
                    if gather:
                        if idx_stride:
                            for tt in range(CHUNK // 16):
                                sl = pl.ds(tt * 16, 16)
                                ibuf[0, sl] = ibuf[0, sl] + off
                        pltpu.async_copy(tbl_ref.at[sidx(0)], rows.at[1],
                                         gsem[0]).wait()
                        pltpu.sync_copy(rows.at[1], acc.at[didx(0)], add=True)
                    else:
                        pltpu.sync_copy(rows.at[0], acc.at[didx(0)], add=True)

        plsc.subcore_barrier()
        pltpu.sync_copy(acc.at[pl.ds(s * RPT, RPT), :],
                        out_ref.at[pl.ds(c * N + s * RPT, RPT), :])

    return pl.kernel(
        body,
        out_type=jax.ShapeDtypeStruct((NC * N, F), jnp.float32),
        mesh=plsc.VectorSubcoreMesh(core_axis_name="c", subcore_axis_name="s",
                                    num_cores=NC, num_subcores=NT),
        compiler_params=pltpu.CompilerParams(use_tc_tiling_on_sc=False),
        scratch_types=[
            pltpu.VMEM((U, 2 * CHUNK), jnp.int32),
            pltpu.VMEM((U, CHUNK, F), jnp.float32),
            pltpu.VMEM((ZROWS, F), jnp.float32),
            pltpu.VMEM_SHARED((N, F), jnp.float32),
        ] + [pltpu.SemaphoreType.DMA] * (2 * U),
    )


def _deg_pass(ei, t):
    return _sc_pass(16, gather=False, split_edges=True, idx_stride=0)(ei, t)


def _agg1_pass(ei, t):
    return _sc_pass(16, gather=True, split_edges=True, idx_stride=0)(ei, t)


def _agg2_pass(ei, t):
    return _sc_pass(32, gather=True, split_edges=False, idx_stride=N)(ei, t)


def _agg3_pass(ei, t):
    return _sc_pass(16, gather=True, split_edges=False, idx_stride=N)(ei, t)


def _bn_st(g, b, rm, rv):
    s = g * lax.rsqrt(rv + 1e-5)
    return s, b - rm * s


def _tc1_body(d0, d1, x16, dis_o, y1_o):
    disv = lax.rsqrt(d0[...] + d1[...] + 1.0)
    dis_o[...] = disv[:, :1]
    y1_o[...] = disv * x16[...]


def _tc2_body(p0, p1, x16, dis, w1, b1, g, bb, rm, rv, w2h, y2_o):
    dis_v = dis[...]
    aggx = dis_v * (p0[...] + p1[...]) + (dis_v * dis_v) * x16[...]
    z = jnp.dot(aggx, w1[...], preferred_element_type=jnp.float32) + b1[...]
    s, t = _bn_st(g[...], bb[...], rm[...], rv[...])
    h1 = jnp.maximum(z * s + t, 0.0)
    y2_o[0] = dis_v * jnp.dot(h1, w2h[0], preferred_element_type=jnp.float32)


def _tc3_body(a2a, a2b, y2, dis, b2, g, bb, rm, rv, w3h, y3_o):
    dis_v = dis[...]
    conv = jnp.concatenate(
        [dis_v * (a2a[...] + y2[0]), dis_v * (a2b[...] + y2[1])], axis=1)
    conv = conv + b2[...]
    s, t = _bn_st(g[...], bb[...], rm[...], rv[...])
    h2 = jnp.maximum(conv * s + t, 0.0)
    y3_o[0] = dis_v * jnp.dot(h2, w3h[0], preferred_element_type=jnp.float32)


def _tc4_body(a3a, a3b, y3, dis, x16, b3, g, bb, rm, rv,
              cp_w1, cp_b1, cp_w2, cp_b2, ci_w, ci_b, cls_w, cls_b, out_o):
    dis_v = dis[...]
    conv = jnp.concatenate(
        [dis_v * (a3a[...] + y3[0]), dis_v * (a3b[...] + y3[1])], axis=1)
    conv = conv + b3[...]
    s, t = _bn_st(g[...], bb[...], rm[...], rv[...])
    h3 = jnp.maximum(conv * s + t, 0.0)
    ch = x16[:, 8:9]
    pc = jnp.maximum(ch * cp_w1[...] + cp_b1[...], 0.0)
    pc = jnp.dot(pc, cp_w2[...], preferred_element_type=jnp.float32) + cp_b2[...]
    ci = ci_w[...]
    hh = (jnp.dot(h3, ci[:32, :], preferred_element_type=jnp.float32)
          + jnp.dot(pc, ci[32:, :], preferred_element_type=jnp.float32)
          + ci_b[...])
    hh = jnp.maximum(hh, 0.0)
    logit = jnp.dot(hh, cls_w[...], preferred_element_type=jnp.float32) + cls_b[...]
    out_o[...] = jax.nn.sigmoid(logit)


def _row_spec(fdim, offset_blocks=0):
    return pl.BlockSpec((B, fdim), lambda i, *_: (i + offset_blocks, 0))


def _full_spec(shape):
    return pl.BlockSpec(shape, lambda i, *_: tuple(0 for _ in shape))


def _build_tc(interpret=False):
    tc1 = pl.pallas_call(
        _tc1_body,
        grid=(NB,),
        in_specs=[_row_spec(16), _row_spec(16, NB), _row_spec(16)],
        out_specs=[_row_spec(1), _row_spec(16)],
        out_shape=[jax.ShapeDtypeStruct((N, 1), jnp.float32),
                   jax.ShapeDtypeStruct((N, 16), jnp.float32)],
        interpret=interpret,
    )
    tc2 = pl.pallas_call(
        _tc2_body,
        grid=(NB, 2),
        in_specs=[_row_spec(16), _row_spec(16, NB), _row_spec(16), _row_spec(1),
                  _full_spec((16, 128)), _full_spec((1, 128)),
                  _full_spec((1, 128)), _full_spec((1, 128)),
                  _full_spec((1, 128)), _full_spec((1, 128)),
                  pl.BlockSpec((1, 128, 32), lambda i, j: (j, 0, 0))],
        out_specs=pl.BlockSpec((1, B, 32), lambda i, j: (j, i, 0)),
        out_shape=jax.ShapeDtypeStruct((2, N, 32), jnp.float32),
        interpret=interpret,
    )
    tc3 = pl.pallas_call(
        _tc3_body,
        grid=(NB, 2),
        in_specs=[_row_spec(32), _row_spec(32, NB),
                  pl.BlockSpec((2, B, 32), lambda i, j: (0, i, 0)),
                  _row_spec(1), _full_spec((1, 64)),
                  _full_spec((1, 64)), _full_spec((1, 64)),
                  _full_spec((1, 64)), _full_spec((1, 64)),
                  pl.BlockSpec((1, 64, 16), lambda i, j: (j, 0, 0))],
        out_specs=pl.BlockSpec((1, B, 16), lambda i, j: (j, i, 0)),
        out_shape=jax.ShapeDtypeStruct((2, N, 16), jnp.float32),
        interpret=interpret,
    )
    tc4 = pl.pallas_call(
        _tc4_body,
        grid=(NB,),
        in_specs=[_row_spec(16), _row_spec(16, NB),
                  pl.BlockSpec((2, B, 16), lambda i: (0, i, 0)),
                  _row_spec(1), _row_spec(16), _full_spec((1, 32)),
                  _full_spec((1, 32)), _full_spec((1, 32)),
                  _full_spec((1, 32)), _full_spec((1, 32)),
                  _full_spec((1, 32)), _full_spec((1, 32)),
                  _full_spec((32, 16)), _full_spec((1, 16)),
                  _full_spec((48, 32)), _full_spec((1, 32)),
                  _full_spec((32, 1)), _full_spec((1, 1))],
        out_specs=_row_spec(1),
        out_shape=jax.ShapeDtypeStruct((N, 1), jnp.float32),
        interpret=interpret,
    )
    return tc1, tc2, tc3, tc4


_tc1, _tc2, _tc3, _tc4 = _build_tc()


def kernel(x, edge_index, cp_w1, cp_b1, cp_w2, cp_b2, w1, b1, w2, b2, w3, b3,
           bn1_g, bn1_b, bn1_rm, bn1_rv, bn2_g, bn2_b, bn2_rm, bn2_rv,
           bn3_g, bn3_b, bn3_rm, bn3_rv, ci_w, ci_b, cls_w, cls_b):
    ei2 = (edge_index.reshape(2, ECH, CHUNK)
           .transpose(1, 0, 2).reshape(ECH, 2 * CHUNK))
    x16 = jnp.pad(x, ((0, 0), (0, 6)))
    w1p = jnp.pad(w1, ((0, 6), (0, 0)))
    r2 = lambda v: v.reshape(1, -1)
    _colsplit = lambda w: w.reshape(w.shape[0], 2, -1).transpose(1, 0, 2)

    deg = _deg_pass(ei2, x16)
    dis, y1 = _tc1(deg, deg, x16)
    l1 = _agg1_pass(ei2, y1)
    y2 = _tc2(l1, l1, x16, dis, w1p, r2(b1), r2(bn1_g), r2(bn1_b),
              r2(bn1_rm), r2(bn1_rv), _colsplit(w2))
    l2 = _agg2_pass(ei2, y2.reshape(NC * N, 32))
    y3 = _tc3(l2, l2, y2, dis, r2(b2), r2(bn2_g), r2(bn2_b),
              r2(bn2_rm), r2(bn2_rv), _colsplit(w3))
    l3 = _agg3_pass(ei2, y3.reshape(NC * N, 16))
    out = _tc4(l3, l3, y3, dis, x16, r2(b3), r2(bn3_g), r2(bn3_b),
               r2(bn3_rm), r2(bn3_rv), cp_w1, r2(cp_b1), cp_w2, r2(cp_b2),
               ci_w, r2(ci_b), cls_w, r2(cls_b))
    return out


# speedup vs baseline: 29.2626x; 1.0776x over previous
import functools

import jax
import jax.numpy as jnp
from jax import lax
from jax.experimental import pallas as pl
from jax.experimental.pallas import tpu as pltpu
from jax.experimental.pallas import tpu_sc as plsc

N = 50000
E = 800000
CHUNK = 128
ECH = E // CHUNK
NC = 2
NT = 16
RPT = N // NT
ZROWS = 125
B = 5000
NB = N // B


@functools.cache
def _sc_pass(F, gather, split_edges, idx_stride):
    U = 6 if F > 16 else 8
    UH = U // 2
    cpc = ECH // NC if split_edges else ECH
    groups, leftover = divmod(cpc, U)
    qg, rg = divmod(groups, NT)

    def body(ei_ref, tbl_ref, out_ref, ibuf, rows, zbuf, acc, *sems):
        gsem, ssem = sems[:U], sems[U:]
        c = lax.axis_index("c")
        s = lax.axis_index("s")
        off = c * idx_stride

        def zrow(i, carry):
            for t in range(F // 16):
                zbuf[i, pl.ds(t * 16, 16)] = jnp.zeros((16,), jnp.float32)
            return carry
        lax.fori_loop(0, ZROWS, zrow, 0)
        if not gather:
            def orow(i, carry):
                for t in range(F // 16):
                    rows[0, i, pl.ds(t * 16, 16)] = jnp.ones((16,), jnp.float32)
                return carry
            lax.fori_loop(0, CHUNK, orow, 0)

        def zcp(k, carry):
            pltpu.sync_copy(zbuf, acc.at[pl.ds(s * RPT + k * ZROWS, ZROWS), :])
            return carry
        lax.fori_loop(0, RPT // ZROWS, zcp, 0)
        plsc.subcore_barrier()

        base_c = c * cpc if split_edges else 0
        gcnt = qg + jnp.where(s < rg, 1, 0)
        gstart = base_c + (s * qg + jnp.minimum(s, rg)) * U

        def sidx(u):
            return ibuf.at[u, pl.ds(0, CHUNK)]

        def didx(u):
            return ibuf.at[u, pl.ds(CHUNK, CHUNK)]

        def scat_desc(u):
            return pltpu.make_async_copy(
                rows.at[u], acc.at[didx(u)], ssem[u])

        def do_group(chunk0, drain):
            for h in range(U // UH):
                u0 = h * UH
                if drain:
                    for u in range(u0, u0 + UH):
                        scat_desc(u).wait()
                pltpu.sync_copy(ei_ref.at[pl.ds(chunk0 + u0, UH), :],
                                ibuf.at[pl.ds(u0, UH), :])
                if gather and idx_stride:
                    for u in range(u0, u0 + UH):
                        for t in range(CHUNK // 16):
                            sl = pl.ds(t * 16, 16)
                            ibuf[u, sl] = ibuf[u, sl] + off
                if gather:
                    for u in range(u0, u0 + UH):
                        pltpu.async_copy(tbl_ref.at[sidx(u)], rows.at[u],
                                         gsem[u])
            for u in range(U):
                if gather:
                    pltpu.make_async_copy(tbl_ref.at[sidx(u)], rows.at[u],
                                          gsem[u]).wait()
                    src = rows.at[u]
                else:
                    src = rows.at[0]
                pltpu.async_copy(src, acc.at[didx(u)], ssem[u], add=True)

        do_group(gstart, drain=False)

        def grp(i, carry):
            do_group(gstart + i * U, drain=True)
            return carry
        lax.fori_loop(1, gcnt, grp, 0)
        for u in range(U):
            if gather:
                pltpu.make_async_copy(rows.at[u], acc.at[didx(u)],
                                      ssem[u]).wait()
            else:
                pltpu.make_async_copy(rows.at[0], acc.at[didx(u)],
                                      ssem[u]).wait()

        if leftover:
            @pl.when(s == NT - 1)
            def _tail():
                for t in range(leftover):
                    ch = base_c + groups * U + t
                    pltpu.sync_copy(ei_ref.at[pl.ds(ch, 1), :],
                                    ibuf.at[pl.ds(0, 1), :])
                    if gather:
                        if idx_stride:
                            for tt in range(CHUNK // 16):
                                sl = pl.ds(tt * 16, 16)
                                ibuf[0, sl] = ibuf[0, sl] + off
                        pltpu.async_copy(tbl_ref.at[sidx(0)], rows.at[1],
                                         gsem[0]).wait()
                        pltpu.sync_copy(rows.at[1], acc.at[didx(0)], add=True)
                    else:
                        pltpu.sync_copy(rows.at[0], acc.at[didx(0)], add=True)

        plsc.subcore_barrier()
        pltpu.sync_copy(acc.at[pl.ds(s * RPT, RPT), :],
                        out_ref.at[pl.ds(c * N + s * RPT, RPT), :])

    return pl.kernel(
        body,
        out_type=jax.ShapeDtypeStruct((NC * N, F), jnp.float32),
        mesh=plsc.VectorSubcoreMesh(core_axis_name="c", subcore_axis_name="s",
                                    num_cores=NC, num_subcores=NT),
        compiler_params=pltpu.CompilerParams(use_tc_tiling_on_sc=False),
        scratch_types=[
            pltpu.VMEM((U, 2 * CHUNK), jnp.int32),
            pltpu.VMEM((U, CHUNK, F), jnp.float32),
            pltpu.VMEM((ZROWS, F), jnp.float32),
            pltpu.VMEM_SHARED((N, F), jnp.float32),
        ] + [pltpu.SemaphoreType.DMA] * (2 * U),
    )


def _deg_pass(ei, t):
    return _sc_pass(16, gather=False, split_edges=True, idx_stride=0)(ei, t)


def _agg1_pass(ei, t):
    return _sc_pass(16, gather=True, split_edges=True, idx_stride=0)(ei, t)


def _agg2_pass(ei, t):
    return _sc_pass(32, gather=True, split_edges=False, idx_stride=N)(ei, t)


def _agg3_pass(ei, t):
    return _sc_pass(16, gather=True, split_edges=False, idx_stride=N)(ei, t)


def _bn_st(g, b, rm, rv):
    s = g * lax.rsqrt(rv + 1e-5)
    return s, b - rm * s


def _tc1_body(d0, d1, x16, dis_o, y1_o):
    disv = lax.rsqrt(d0[...] + d1[...] + 1.0)
    dis_o[...] = disv[:, :1]
    y1_o[...] = disv * x16[...]


def _tc2_body(p0, p1, x16, dis, w1, b1, g, bb, rm, rv, w2h, y2_o):
    dis_v = dis[...]
    aggx = dis_v * (p0[...] + p1[...]) + (dis_v * dis_v) * x16[...]
    z = jnp.dot(aggx, w1[...], preferred_element_type=jnp.float32) + b1[...]
    s, t = _bn_st(g[...], bb[...], rm[...], rv[...])
    h1 = jnp.maximum(z * s + t, 0.0)
    y2_o[0] = dis_v * jnp.dot(h1, w2h[0], preferred_element_type=jnp.float32)


def _tc3_body(a2a, a2b, y2, dis, b2, g, bb, rm, rv, w3h, y3_o):
    dis_v = dis[...]
    conv = jnp.concatenate(
        [dis_v * (a2a[...] + y2[0]), dis_v * (a2b[...] + y2[1])], axis=1)
    conv = conv + b2[...]
    s, t = _bn_st(g[...], bb[...], rm[...], rv[...])
    h2 = jnp.maximum(conv * s + t, 0.0)
    y3_o[0] = dis_v * jnp.dot(h2, w3h[0], preferred_element_type=jnp.float32)


def _tc4_body(a3a, a3b, y3, dis, x16, b3, g, bb, rm, rv,
              cp_w1, cp_b1, cp_w2, cp_b2, ci_w, ci_b, cls_w, cls_b, out_o):
    dis_v = dis[...]
    conv = jnp.concatenate(
        [dis_v * (a3a[...] + y3[0]), dis_v * (a3b[...] + y3[1])], axis=1)
    conv = conv + b3[...]
    s, t = _bn_st(g[...], bb[...], rm[...], rv[...])
    h3 = jnp.maximum(conv * s + t, 0.0)
    ch = x16[:, 8:9]
    pc = jnp.maximum(ch * cp_w1[...] + cp_b1[...], 0.0)
    pc = jnp.dot(pc, cp_w2[...], preferred_element_type=jnp.float32) + cp_b2[...]
    ci = ci_w[...]
    hh = (jnp.dot(h3, ci[:32, :], preferred_element_type=jnp.float32)
          + jnp.dot(pc, ci[32:, :], preferred_element_type=jnp.float32)
          + ci_b[...])
    hh = jnp.maximum(hh, 0.0)
    logit = jnp.dot(hh, cls_w[...], preferred_element_type=jnp.float32) + cls_b[...]
    out_o[...] = jax.nn.sigmoid(logit)


def _row_spec(fdim, offset_blocks=0):
    return pl.BlockSpec((B, fdim), lambda i, *_: (i + offset_blocks, 0))


def _full_spec(shape):
    return pl.BlockSpec(shape, lambda i, *_: tuple(0 for _ in shape))


def _build_tc(interpret=False):
    tc1 = pl.pallas_call(
        _tc1_body,
        grid=(NB,),
        in_specs=[_row_spec(16), _row_spec(16, NB), _row_spec(16)],
        out_specs=[_row_spec(1), _row_spec(16)],
        out_shape=[jax.ShapeDtypeStruct((N, 1), jnp.float32),
                   jax.ShapeDtypeStruct((N, 16), jnp.float32)],
        interpret=interpret,
    )
    tc2 = pl.pallas_call(
        _tc2_body,
        grid=(NB, 2),
        in_specs=[_row_spec(16), _row_spec(16, NB), _row_spec(16), _row_spec(1),
                  _full_spec((16, 128)), _full_spec((1, 128)),
                  _full_spec((1, 128)), _full_spec((1, 128)),
                  _full_spec((1, 128)), _full_spec((1, 128)),
                  pl.BlockSpec((1, 128, 32), lambda i, j: (j, 0, 0))],
        out_specs=pl.BlockSpec((1, B, 32), lambda i, j: (j, i, 0)),
        out_shape=jax.ShapeDtypeStruct((2, N, 32), jnp.float32),
        interpret=interpret,
    )
    tc3 = pl.pallas_call(
        _tc3_body,
        grid=(NB, 2),
        in_specs=[_row_spec(32), _row_spec(32, NB),
                  pl.BlockSpec((2, B, 32), lambda i, j: (0, i, 0)),
                  _row_spec(1), _full_spec((1, 64)),
                  _full_spec((1, 64)), _full_spec((1, 64)),
                  _full_spec((1, 64)), _full_spec((1, 64)),
                  pl.BlockSpec((1, 64, 16), lambda i, j: (j, 0, 0))],
        out_specs=pl.BlockSpec((1, B, 16), lambda i, j: (j, i, 0)),
        out_shape=jax.ShapeDtypeStruct((2, N, 16), jnp.float32),
        interpret=interpret,
    )
    tc4 = pl.pallas_call(
        _tc4_body,
        grid=(NB,),
        in_specs=[_row_spec(16), _row_spec(16, NB),
                  pl.BlockSpec((2, B, 16), lambda i: (0, i, 0)),
                  _row_spec(1), _row_spec(16), _full_spec((1, 32)),
                  _full_spec((1, 32)), _full_spec((1, 32)),
                  _full_spec((1, 32)), _full_spec((1, 32)),
                  _full_spec((1, 32)), _full_spec((1, 32)),
                  _full_spec((32, 16)), _full_spec((1, 16)),
                  _full_spec((48, 32)), _full_spec((1, 32)),
                  _full_spec((32, 1)), _full_spec((1, 1))],
        out_specs=_row_spec(1),
        out_shape=jax.ShapeDtypeStruct((N, 1), jnp.float32),
        interpret=interpret,
    )
    return tc1, tc2, tc3, tc4


_tc1, _tc2, _tc3, _tc4 = _build_tc()


def kernel(x, edge_index, cp_w1, cp_b1, cp_w2, cp_b2, w1, b1, w2, b2, w3, b3,
           bn1_g, bn1_b, bn1_rm, bn1_rv, bn2_g, bn2_b, bn2_rm, bn2_rv,
           bn3_g, bn3_b, bn3_rm, bn3_rv, ci_w, ci_b, cls_w, cls_b):
    ei2 = (edge_index.reshape(2, ECH, CHUNK)
           .transpose(1, 0, 2).reshape(ECH, 2 * CHUNK))
    x16 = jnp.pad(x, ((0, 0), (0, 6)))
    w1p = jnp.pad(w1, ((0, 6), (0, 0)))
    r2 = lambda v: v.reshape(1, -1)
    _colsplit = lambda w: w.reshape(w.shape[0], 2, -1).transpose(1, 0, 2)

    deg = _deg_pass(ei2, x16)
    dis, y1 = _tc1(deg, deg, x16)
    l1 = _agg1_pass(ei2, y1)
    y2 = _tc2(l1, l1, x16, dis, w1p, r2(b1), r2(bn1_g), r2(bn1_b),
              r2(bn1_rm), r2(bn1_rv), _colsplit(w2))
    l2 = _agg2_pass(ei2, y2.reshape(NC * N, 32))
    y3 = _tc3(l2, l2, y2, dis, r2(b2), r2(bn2_g), r2(bn2_b),
              r2(bn2_rm), r2(bn2_rv), _colsplit(w3))
    l3 = _agg3_pass(ei2, y3.reshape(NC * N, 16))
    out = _tc4(l3, l3, y3, dis, x16, r2(b3), r2(bn3_g), r2(bn3_b),
               r2(bn3_rm), r2(bn3_rv), cp_w1, r2(cp_b1), cp_w2, r2(cp_b2),
               ci_w, r2(ci_b), cls_w, r2(cls_b))
    return out


# speedup vs baseline: 46.5615x; 1.5912x over previous
import functools

import jax
import jax.numpy as jnp
from jax import lax
from jax.experimental import pallas as pl
from jax.experimental.pallas import tpu as pltpu
from jax.experimental.pallas import tpu_sc as plsc

N = 50000
N_PAD = 51200
E = 800000
CHUNK = 128
ECH = E // CHUNK
NC = 2
NT = 16
RPT = N // NT
ZROWS = 125


@functools.cache
def _sc_pass(F, gather, split_edges, idx_stride):
    U = 6 if F > 16 else 8
    UH = U // 2
    cpc = ECH // NC if split_edges else ECH
    groups, leftover = divmod(cpc, U)
    qg, rg = divmod(groups, NT)

    def body(ei_ref, tbl_ref, out_ref, ibuf, rows, zbuf, acc, *sems):
        gsem, ssem = sems[:U], sems[U:]
        c = lax.axis_index("c")
        s = lax.axis_index("s")
        off = c * idx_stride

        def zrow(i, carry):
            for t in range(F // 16):
                zbuf[i, pl.ds(t * 16, 16)] = jnp.zeros((16,), jnp.float32)
            return carry
        lax.fori_loop(0, ZROWS, zrow, 0)
        if not gather:
            def orow(i, carry):
                for t in range(F // 16):
                    rows[0, i, pl.ds(t * 16, 16)] = jnp.ones((16,), jnp.float32)
                return carry
            lax.fori_loop(0, CHUNK, orow, 0)

        def zcp(k, carry):
            pltpu.sync_copy(zbuf, acc.at[pl.ds(s * RPT + k * ZROWS, ZROWS), :])
            return carry
        lax.fori_loop(0, RPT // ZROWS, zcp, 0)
        plsc.subcore_barrier()

        base_c = c * cpc if split_edges else 0
        gcnt = qg + jnp.where(s < rg, 1, 0)
        gstart = base_c + (s * qg + jnp.minimum(s, rg)) * U

        def sidx(u):
            return ibuf.at[u, pl.ds(0, CHUNK)]

        def didx(u):
            return ibuf.at[u, pl.ds(CHUNK, CHUNK)]

        def scat_desc(u):
            return pltpu.make_async_copy(
                rows.at[u], acc.at[didx(u)], ssem[u])

        def do_group(chunk0, drain):
            for h in range(U // UH):
                u0 = h * UH
                if drain:
                    for u in range(u0, u0 + UH):
                        scat_desc(u).wait()
                pltpu.sync_copy(ei_ref.at[pl.ds(chunk0 + u0, UH), :],
                                ibuf.at[pl.ds(u0, UH), :])
                if gather and idx_stride:
                    for u in range(u0, u0 + UH):
                        for t in range(CHUNK // 16):
                            sl = pl.ds(t * 16, 16)
                            ibuf[u, sl] = ibuf[u, sl] + off
                if gather:
                    for u in range(u0, u0 + UH):
                        pltpu.async_copy(tbl_ref.at[sidx(u)], rows.at[u],
                                         gsem[u])
            for u in range(U):
                if gather:
                    pltpu.make_async_copy(tbl_ref.at[sidx(u)], rows.at[u],
                                          gsem[u]).wait()
                    src = rows.at[u]
                else:
                    src = rows.at[0]
                pltpu.async_copy(src, acc.at[didx(u)], ssem[u], add=True)

        do_group(gstart, drain=False)

        def grp(i, carry):
            do_group(gstart + i * U, drain=True)
            return carry
        lax.fori_loop(1, gcnt, grp, 0)
        for u in range(U):
            if gather:
                pltpu.make_async_copy(rows.at[u], acc.at[didx(u)],
                                      ssem[u]).wait()
            else:
                pltpu.make_async_copy(rows.at[0], acc.at[didx(u)],
                                      ssem[u]).wait()

        if leftover:
            @pl.when(s == NT - 1)
            def _tail():
                for t in range(leftover):
                    ch = base_c + groups * U + t
                    pltpu.sync_copy(ei_ref.at[pl.ds(ch, 1), :],
                                    ibuf.at[pl.ds(0, 1), :])
                    if gather:
                        if idx_stride:
                            for tt in range(CHUNK // 16):
                                sl = pl.ds(tt * 16, 16)
                                ibuf[0, sl] = ibuf[0, sl] + off
                        pltpu.async_copy(tbl_ref.at[sidx(0)], rows.at[1],
                                         gsem[0]).wait()
                        pltpu.sync_copy(rows.at[1], acc.at[didx(0)], add=True)
                    else:
                        pltpu.sync_copy(rows.at[0], acc.at[didx(0)], add=True)

        plsc.subcore_barrier()
        pltpu.sync_copy(acc.at[pl.ds(s * RPT, RPT), :],
                        out_ref.at[pl.ds(c * N_PAD + s * RPT, RPT), :])

    return pl.kernel(
        body,
        out_type=jax.ShapeDtypeStruct((NC * N_PAD, F), jnp.float32),
        mesh=plsc.VectorSubcoreMesh(core_axis_name="c", subcore_axis_name="s",
                                    num_cores=NC, num_subcores=NT),
        compiler_params=pltpu.CompilerParams(use_tc_tiling_on_sc=False),
        scratch_types=[
            pltpu.VMEM((U, 2 * CHUNK), jnp.int32),
            pltpu.VMEM((U, CHUNK, F), jnp.float32),
            pltpu.VMEM((ZROWS, F), jnp.float32),
            pltpu.VMEM_SHARED((N, F), jnp.float32),
        ] + [pltpu.SemaphoreType.DMA] * (2 * U),
    )


def _deg_pass(ei, t):
    return _sc_pass(16, gather=False, split_edges=True, idx_stride=0)(ei, t)


def _agg1_pass(ei, t):
    return _sc_pass(16, gather=True, split_edges=True, idx_stride=0)(ei, t)


def _agg2_pass(ei, t):
    return _sc_pass(32, gather=True, split_edges=False, idx_stride=N_PAD)(ei, t)


def _agg3_pass(ei, t):
    return _sc_pass(16, gather=True, split_edges=False, idx_stride=N_PAD)(ei, t)


NP = N_PAD // 8
BP = 1280
NBP = NP // BP


def _bn_st(g, b, rm, rv):
    s = g * lax.rsqrt(rv + 1e-5)
    return s, b - rm * s


def _tc1_body(degr, xv, gmat, dis_o, y1_o):
    disv = lax.rsqrt(degr[0] + degr[1] + 1.0)
    dis_o[...] = disv
    y1_o[...] = disv * jnp.dot(xv[...], gmat[...],
                               preferred_element_type=jnp.float32)


def _tc2_body(l1r, xv, disP, gmat, w1bd, b1t, g, bb, rm, rv, s32, w2bd, y2_o):
    d = disP[...]
    x16 = jnp.dot(xv[...], gmat[...], preferred_element_type=jnp.float32)
    aggx = d * (l1r[0] + l1r[1]) + (d * d) * x16
    z = jnp.dot(aggx, w1bd[...], preferred_element_type=jnp.float32) + b1t[...]
    sc, sh = _bn_st(g[...], bb[...], rm[...], rv[...])
    h1 = jnp.maximum(z * sc + sh, 0.0)
    d32 = jnp.dot(d, s32[...], preferred_element_type=jnp.float32)
    y2_o[0] = d32 * jnp.dot(h1, w2bd[0], preferred_element_type=jnp.float32)


def _tc3_body(l2r, y2s, disP, s32, b2pt, g, bb, rm, rv, w3bd, y3_o):
    d = disP[...]
    d32 = jnp.dot(d, s32[...], preferred_element_type=jnp.float32)
    conv = jnp.concatenate([d32 * (l2r[0] + y2s[0]),
                            d32 * (l2r[1] + y2s[1])], axis=1) + b2pt[...]
    sc, sh = _bn_st(g[...], bb[...], rm[...], rv[...])
    h2 = jnp.maximum(conv * sc + sh, 0.0)
    y3_o[0] = d * jnp.dot(h2, w3bd[0], preferred_element_type=jnp.float32)


def _tc4_body(l3r, y3s, disP, xv, gmat, chsel, b3pt, g, bb, rm, rv,
              cpw1t, cpb1t, cpw2bd, cpb2t, cibd, cibd2, cibt, clsbd, clsbt,
              out_o):
    d = disP[...]
    conv = jnp.concatenate([d * (l3r[0] + y3s[0]),
                            d * (l3r[1] + y3s[1])], axis=1) + b3pt[...]
    sc, sh = _bn_st(g[...], bb[...], rm[...], rv[...])
    h3 = jnp.maximum(conv * sc + sh, 0.0)
    x16 = jnp.dot(xv[...], gmat[...], preferred_element_type=jnp.float32)
    chP = jnp.dot(x16, chsel[...], preferred_element_type=jnp.float32)
    pc1 = jnp.maximum(chP * cpw1t[...] + cpb1t[...], 0.0)
    pc = jnp.dot(pc1, cpw2bd[...],
                 preferred_element_type=jnp.float32) + cpb2t[...]
    hh = (jnp.dot(h3, cibd[...], preferred_element_type=jnp.float32)
          + jnp.dot(pc, cibd2[...], preferred_element_type=jnp.float32)
          + cibt[...])
    hh = jnp.maximum(hh, 0.0)
    logit = jnp.dot(hh, clsbd[...],
                    preferred_element_type=jnp.float32) + clsbt[...]
    out_o[...] = jax.nn.sigmoid(logit)


def _prow(fdim):
    return pl.BlockSpec((BP, fdim), lambda i, *_: (i, 0))


def _pfull(shape):
    return pl.BlockSpec(shape, lambda *_: tuple(0 for _ in shape))


def _p2(fdim):
    return pl.BlockSpec((2, BP, fdim), lambda i, *_: (0, i, 0))


def _build_tc(interpret=False):
    tc1 = pl.pallas_call(
        _tc1_body,
        grid=(NBP,),
        in_specs=[_p2(128), _prow(80), _pfull((80, 128))],
        out_specs=[_prow(128), _prow(128)],
        out_shape=[jax.ShapeDtypeStruct((NP, 128), jnp.float32),
                   jax.ShapeDtypeStruct((NP, 128), jnp.float32)],
        interpret=interpret,
    )
    tc2 = pl.pallas_call(
        _tc2_body,
        grid=(NBP, 2),
        in_specs=[_p2(128), _prow(80), _prow(128), _pfull((80, 128)),
                  _pfull((128, 1024)), _pfull((1, 1024)),
                  _pfull((1, 1024)), _pfull((1, 1024)),
                  _pfull((1, 1024)), _pfull((1, 1024)),
                  _pfull((128, 256)),
                  pl.BlockSpec((1, 1024, 256), lambda i, j: (j, 0, 0))],
        out_specs=pl.BlockSpec((1, BP, 256), lambda i, j: (j, i, 0)),
        out_shape=jax.ShapeDtypeStruct((2, NP, 256), jnp.float32),
        interpret=interpret,
    )
    tc3 = pl.pallas_call(
        _tc3_body,
        grid=(NBP, 2),
        in_specs=[_p2(256), _p2(256), _prow(128), _pfull((128, 256)),
                  _pfull((1, 512)),
                  _pfull((1, 512)), _pfull((1, 512)),
                  _pfull((1, 512)), _pfull((1, 512)),
                  pl.BlockSpec((1, 512, 128), lambda i, j: (j, 0, 0))],
        out_specs=pl.BlockSpec((1, BP, 128), lambda i, j: (j, i, 0)),
        out_shape=jax.ShapeDtypeStruct((2, NP, 128), jnp.float32),
        interpret=interpret,
    )
    tc4 = pl.pallas_call(
        _tc4_body,
        grid=(NBP,),
        in_specs=[_p2(128), _p2(128), _prow(128), _prow(80),
                  _pfull((80, 128)), _pfull((128, 256)), _pfull((1, 256)),
                  _pfull((1, 256)), _pfull((1, 256)),
                  _pfull((1, 256)), _pfull((1, 256)),
                  _pfull((1, 256)), _pfull((1, 256)),
                  _pfull((256, 128)), _pfull((1, 128)),
                  _pfull((256, 256)), _pfull((128, 256)), _pfull((1, 256)),
                  _pfull((256, 8)), _pfull((1, 8))],
        out_specs=_prow(8),
        out_shape=jax.ShapeDtypeStruct((NP, 8), jnp.float32),
        interpret=interpret,
    )
    return tc1, tc2, tc3, tc4


_tc1, _tc2, _tc3, _tc4 = _build_tc()


def kernel(x, edge_index, cp_w1, cp_b1, cp_w2, cp_b2, w1, b1, w2, b2, w3, b3,
           bn1_g, bn1_b, bn1_rm, bn1_rv, bn2_g, bn2_b, bn2_rm, bn2_rv,
           bn3_g, bn3_b, bn3_rm, bn3_rv, ci_w, ci_b, cls_w, cls_b):
    f32 = jnp.float32
    ei2 = (edge_index.reshape(2, ECH, CHUNK)
           .transpose(1, 0, 2).reshape(ECH, 2 * CHUNK))
    xv = jnp.pad(x.reshape(N // 8, 80), ((0, NP - N // 8), (0, 0)))
    ey8 = jnp.eye(8, dtype=f32)
    kr = lambda w: jnp.kron(ey8, w)
    t8 = lambda v: jnp.tile(v.reshape(-1), 8).reshape(1, -1)
    pt = lambda v, h: jnp.concatenate([t8(v[:h]), t8(v[h:])], axis=1)

    gmat = kr(jnp.eye(10, 16, dtype=f32))
    s32 = kr(jnp.zeros((16, 32), f32).at[0, :].set(1.0))
    chsel = kr(jnp.zeros((16, 32), f32).at[8, :].set(1.0))
    w1bd = kr(jnp.pad(w1, ((0, 6), (0, 0))))
    w2bds = jnp.stack([kr(w2[:, :32]), kr(w2[:, 32:])])
    w3bds = jnp.stack([
        jnp.concatenate([kr(w3[:32, 16 * j:16 * (j + 1)]),
                         kr(w3[32:, 16 * j:16 * (j + 1)])], axis=0)
        for j in (0, 1)])
    cibd = jnp.concatenate([kr(ci_w[:16]), kr(ci_w[16:32])], axis=0)
    cibd2 = kr(ci_w[32:])
    cpw2bd = kr(cp_w2)
    clsbd = kr(cls_w)

    deg = _deg_pass(ei2, x)
    disP, y1P = _tc1(deg.reshape(2, NP, 128), xv, gmat)
    l1 = _agg1_pass(ei2, y1P.reshape(N_PAD, 16))
    y2s = _tc2(l1.reshape(2, NP, 128), xv, disP, gmat, w1bd, t8(b1),
               t8(bn1_g), t8(bn1_b), t8(bn1_rm), t8(bn1_rv), s32, w2bds)
    l2 = _agg2_pass(ei2, y2s.reshape(NC * N_PAD, 32))
    y3s = _tc3(l2.reshape(2, NP, 256), y2s, disP, s32, pt(b2, 32),
               pt(bn2_g, 32), pt(bn2_b, 32), pt(bn2_rm, 32), pt(bn2_rv, 32),
               w3bds)
    l3 = _agg3_pass(ei2, y3s.reshape(NC * N_PAD, 16))
    outP = _tc4(l3.reshape(2, NP, 128), y3s, disP, xv, gmat, chsel,
                pt(b3, 16), pt(bn3_g, 16), pt(bn3_b, 16), pt(bn3_rm, 16),
                pt(bn3_rv, 16), t8(cp_w1.reshape(-1)), t8(cp_b1), cpw2bd,
                t8(cp_b2), cibd, cibd2, t8(ci_b), clsbd, t8(cls_b))
    return outP.reshape(N_PAD, 1)[:N]


# speedup vs baseline: 49.8738x; 1.0711x over previous
import functools

import jax
import jax.numpy as jnp
from jax import lax
from jax.experimental import pallas as pl
from jax.experimental.pallas import tpu as pltpu
from jax.experimental.pallas import tpu_sc as plsc

N = 50000
N_PAD = 51200
E = 800000
CHUNK = 128
ECH = E // CHUNK
NC = 2
NT = 16
RPT = N // NT
ZROWS = 125


@functools.cache
def _sc_pass(F, gather, split_edges, idx_stride):
    U = 6 if F > 16 else 12
    UH = U // 2
    cpc = ECH // NC if split_edges else ECH
    groups, leftover = divmod(cpc, U)
    qg, rg = divmod(groups, NT)

    def body(ei_ref, tbl_ref, out_ref, ibuf, rows, zbuf, acc, *sems):
        gsem, ssem = sems[:U], sems[U:]
        c = lax.axis_index("c")
        s = lax.axis_index("s")
        off = c * idx_stride

        def zrow(i, carry):
            for t in range(F // 16):
                zbuf[i, pl.ds(t * 16, 16)] = jnp.zeros((16,), jnp.float32)
            return carry
        lax.fori_loop(0, ZROWS, zrow, 0)
        if not gather:
            def orow(i, carry):
                for t in range(F // 16):
                    rows[0, i, pl.ds(t * 16, 16)] = jnp.ones((16,), jnp.float32)
                return carry
            lax.fori_loop(0, CHUNK, orow, 0)

        def zcp(k, carry):
            pltpu.sync_copy(zbuf, acc.at[pl.ds(s * RPT + k * ZROWS, ZROWS), :])
            return carry
        lax.fori_loop(0, RPT // ZROWS, zcp, 0)
        plsc.subcore_barrier()

        base_c = c * cpc if split_edges else 0
        gcnt = qg + jnp.where(s < rg, 1, 0)
        gstart = base_c + (s * qg + jnp.minimum(s, rg)) * U

        def sidx(u):
            return ibuf.at[u, pl.ds(0, CHUNK)]

        def didx(u):
            return ibuf.at[u, pl.ds(CHUNK, CHUNK)]

        def scat_desc(u):
            return pltpu.make_async_copy(
                rows.at[u], acc.at[didx(u)], ssem[u])

        def do_group(chunk0, drain):
            for h in range(U // UH):
                u0 = h * UH
                if drain:
                    for u in range(u0, u0 + UH):
                        scat_desc(u).wait()
                pltpu.sync_copy(ei_ref.at[pl.ds(chunk0 + u0, UH), :],
                                ibuf.at[pl.ds(u0, UH), :])
                if gather and idx_stride:
                    for u in range(u0, u0 + UH):
                        for t in range(CHUNK // 16):
                            sl = pl.ds(t * 16, 16)
                            ibuf[u, sl] = ibuf[u, sl] + off
                if gather:
                    for u in range(u0, u0 + UH):
                        pltpu.async_copy(tbl_ref.at[sidx(u)], rows.at[u],
                                         gsem[u])
            for u in range(U):
                if gather:
                    pltpu.make_async_copy(tbl_ref.at[sidx(u)], rows.at[u],
                                          gsem[u]).wait()
                    src = rows.at[u]
                else:
                    src = rows.at[0]
                pltpu.async_copy(src, acc.at[didx(u)], ssem[u], add=True)

        do_group(gstart, drain=False)

        def grp(i, carry):
            do_group(gstart + i * U, drain=True)
            return carry
        lax.fori_loop(1, gcnt, grp, 0)
        for u in range(U):
            if gather:
                pltpu.make_async_copy(rows.at[u], acc.at[didx(u)],
                                      ssem[u]).wait()
            else:
                pltpu.make_async_copy(rows.at[0], acc.at[didx(u)],
                                      ssem[u]).wait()

        if leftover:
            @pl.when(s == NT - 1)
            def _tail():
                for t in range(leftover):
                    ch = base_c + groups * U + t
                    pltpu.sync_copy(ei_ref.at[pl.ds(ch, 1), :],
                                    ibuf.at[pl.ds(0, 1), :])
                    if gather:
                        if idx_stride:
                            for tt in range(CHUNK // 16):
                                sl = pl.ds(tt * 16, 16)
                                ibuf[0, sl] = ibuf[0, sl] + off
                        pltpu.async_copy(tbl_ref.at[sidx(0)], rows.at[1],
                                         gsem[0]).wait()
                        pltpu.sync_copy(rows.at[1], acc.at[didx(0)], add=True)
                    else:
                        pltpu.sync_copy(rows.at[0], acc.at[didx(0)], add=True)

        plsc.subcore_barrier()
        pltpu.sync_copy(acc.at[pl.ds(s * RPT, RPT), :],
                        out_ref.at[pl.ds(c * N_PAD + s * RPT, RPT), :])

    return pl.kernel(
        body,
        out_type=jax.ShapeDtypeStruct((NC * N_PAD, F), jnp.float32),
        mesh=plsc.VectorSubcoreMesh(core_axis_name="c", subcore_axis_name="s",
                                    num_cores=NC, num_subcores=NT),
        compiler_params=pltpu.CompilerParams(use_tc_tiling_on_sc=False),
        scratch_types=[
            pltpu.VMEM((U, 2 * CHUNK), jnp.int32),
            pltpu.VMEM((U, CHUNK, F), jnp.float32),
            pltpu.VMEM((ZROWS, F), jnp.float32),
            pltpu.VMEM_SHARED((N, F), jnp.float32),
        ] + [pltpu.SemaphoreType.DMA] * (2 * U),
    )


def _deg_pass(ei, t):
    return _sc_pass(16, gather=False, split_edges=True, idx_stride=0)(ei, t)


def _agg1_pass(ei, t):
    return _sc_pass(16, gather=True, split_edges=True, idx_stride=0)(ei, t)


def _agg2_pass(ei, t):
    return _sc_pass(32, gather=True, split_edges=False, idx_stride=N_PAD)(ei, t)


def _agg3_pass(ei, t):
    return _sc_pass(16, gather=True, split_edges=False, idx_stride=N_PAD)(ei, t)


NP = N_PAD // 8
BP = 1280
NBP = NP // BP


def _bn_st(g, b, rm, rv):
    s = g * lax.rsqrt(rv + 1e-5)
    return s, b - rm * s


def _tc1_body(degr, xv, gmat, dis_o, y1_o):
    disv = lax.rsqrt(degr[0] + degr[1] + 1.0)
    dis_o[...] = disv
    y1_o[...] = disv * jnp.dot(xv[...], gmat[...],
                               preferred_element_type=jnp.float32)


def _tc2_body(l1r, xv, disP, gmat, w1bd, b1t, g, bb, rm, rv, s32, w2bd, y2_o):
    d = disP[...]
    x16 = jnp.dot(xv[...], gmat[...], preferred_element_type=jnp.float32)
    aggx = d * (l1r[0] + l1r[1]) + (d * d) * x16
    z = jnp.dot(aggx, w1bd[...], preferred_element_type=jnp.float32) + b1t[...]
    sc, sh = _bn_st(g[...], bb[...], rm[...], rv[...])
    h1 = jnp.maximum(z * sc + sh, 0.0)
    d32 = jnp.dot(d, s32[...], preferred_element_type=jnp.float32)
    y2_o[0] = d32 * jnp.dot(h1, w2bd[0], preferred_element_type=jnp.float32)


def _tc3_body(l2r, y2s, disP, s32, b2pt, g, bb, rm, rv, w3bd, y3_o):
    d = disP[...]
    d32 = jnp.dot(d, s32[...], preferred_element_type=jnp.float32)
    conv = jnp.concatenate([d32 * (l2r[0] + y2s[0]),
                            d32 * (l2r[1] + y2s[1])], axis=1) + b2pt[...]
    sc, sh = _bn_st(g[...], bb[...], rm[...], rv[...])
    h2 = jnp.maximum(conv * sc + sh, 0.0)
    y3_o[0] = d * jnp.dot(h2, w3bd[0], preferred_element_type=jnp.float32)


def _tc4_body(l3r, y3s, disP, xv, gmat, chsel, b3pt, g, bb, rm, rv,
              cpw1t, cpb1t, cpw2bd, cpb2t, cibd, cibd2, cibt, clsbd, clsbt,
              out_o):
    d = disP[...]
    conv = jnp.concatenate([d * (l3r[0] + y3s[0]),
                            d * (l3r[1] + y3s[1])], axis=1) + b3pt[...]
    sc, sh = _bn_st(g[...], bb[...], rm[...], rv[...])
    h3 = jnp.maximum(conv * sc + sh, 0.0)
    x16 = jnp.dot(xv[...], gmat[...], preferred_element_type=jnp.float32)
    chP = jnp.dot(x16, chsel[...], preferred_element_type=jnp.float32)
    pc1 = jnp.maximum(chP * cpw1t[...] + cpb1t[...], 0.0)
    pc = jnp.dot(pc1, cpw2bd[...],
                 preferred_element_type=jnp.float32) + cpb2t[...]
    hh = (jnp.dot(h3, cibd[...], preferred_element_type=jnp.float32)
          + jnp.dot(pc, cibd2[...], preferred_element_type=jnp.float32)
          + cibt[...])
    hh = jnp.maximum(hh, 0.0)
    logit = jnp.dot(hh, clsbd[...],
                    preferred_element_type=jnp.float32) + clsbt[...]
    out_o[...] = jax.nn.sigmoid(logit)


def _prow(fdim):
    return pl.BlockSpec((BP, fdim), lambda i, *_: (i, 0))


def _pfull(shape):
    return pl.BlockSpec(shape, lambda *_: tuple(0 for _ in shape))


def _p2(fdim):
    return pl.BlockSpec((2, BP, fdim), lambda i, *_: (0, i, 0))


def _build_tc(interpret=False):
    tc1 = pl.pallas_call(
        _tc1_body,
        grid=(NBP,),
        in_specs=[_p2(128), _prow(80), _pfull((80, 128))],
        out_specs=[_prow(128), _prow(128)],
        out_shape=[jax.ShapeDtypeStruct((NP, 128), jnp.float32),
                   jax.ShapeDtypeStruct((NP, 128), jnp.float32)],
        interpret=interpret,
    )
    tc2 = pl.pallas_call(
        _tc2_body,
        grid=(NBP, 2),
        in_specs=[_p2(128), _prow(80), _prow(128), _pfull((80, 128)),
                  _pfull((128, 1024)), _pfull((1, 1024)),
                  _pfull((1, 1024)), _pfull((1, 1024)),
                  _pfull((1, 1024)), _pfull((1, 1024)),
                  _pfull((128, 256)),
                  pl.BlockSpec((1, 1024, 256), lambda i, j: (j, 0, 0))],
        out_specs=pl.BlockSpec((1, BP, 256), lambda i, j: (j, i, 0)),
        out_shape=jax.ShapeDtypeStruct((2, NP, 256), jnp.float32),
        interpret=interpret,
    )
    tc3 = pl.pallas_call(
        _tc3_body,
        grid=(NBP, 2),
        in_specs=[_p2(256), _p2(256), _prow(128), _pfull((128, 256)),
                  _pfull((1, 512)),
                  _pfull((1, 512)), _pfull((1, 512)),
                  _pfull((1, 512)), _pfull((1, 512)),
                  pl.BlockSpec((1, 512, 128), lambda i, j: (j, 0, 0))],
        out_specs=pl.BlockSpec((1, BP, 128), lambda i, j: (j, i, 0)),
        out_shape=jax.ShapeDtypeStruct((2, NP, 128), jnp.float32),
        interpret=interpret,
    )
    tc4 = pl.pallas_call(
        _tc4_body,
        grid=(NBP,),
        in_specs=[_p2(128), _p2(128), _prow(128), _prow(80),
                  _pfull((80, 128)), _pfull((128, 256)), _pfull((1, 256)),
                  _pfull((1, 256)), _pfull((1, 256)),
                  _pfull((1, 256)), _pfull((1, 256)),
                  _pfull((1, 256)), _pfull((1, 256)),
                  _pfull((256, 128)), _pfull((1, 128)),
                  _pfull((256, 256)), _pfull((128, 256)), _pfull((1, 256)),
                  _pfull((256, 8)), _pfull((1, 8))],
        out_specs=_prow(8),
        out_shape=jax.ShapeDtypeStruct((NP, 8), jnp.float32),
        interpret=interpret,
    )
    return tc1, tc2, tc3, tc4


_tc1, _tc2, _tc3, _tc4 = _build_tc()


def kernel(x, edge_index, cp_w1, cp_b1, cp_w2, cp_b2, w1, b1, w2, b2, w3, b3,
           bn1_g, bn1_b, bn1_rm, bn1_rv, bn2_g, bn2_b, bn2_rm, bn2_rv,
           bn3_g, bn3_b, bn3_rm, bn3_rv, ci_w, ci_b, cls_w, cls_b):
    f32 = jnp.float32
    ei2 = (edge_index.reshape(2, ECH, CHUNK)
           .transpose(1, 0, 2).reshape(ECH, 2 * CHUNK))
    xv = jnp.pad(x.reshape(N // 8, 80), ((0, NP - N // 8), (0, 0)))
    ey8 = jnp.eye(8, dtype=f32)
    kr = lambda w: jnp.kron(ey8, w)
    t8 = lambda v: jnp.tile(v.reshape(-1), 8).reshape(1, -1)
    pt = lambda v, h: jnp.concatenate([t8(v[:h]), t8(v[h:])], axis=1)

    gmat = kr(jnp.eye(10, 16, dtype=f32))
    s32 = kr(jnp.zeros((16, 32), f32).at[0, :].set(1.0))
    chsel = kr(jnp.zeros((16, 32), f32).at[8, :].set(1.0))
    w1bd = kr(jnp.pad(w1, ((0, 6), (0, 0))))
    w2bds = jnp.stack([kr(w2[:, :32]), kr(w2[:, 32:])])
    w3bds = jnp.stack([
        jnp.concatenate([kr(w3[:32, 16 * j:16 * (j + 1)]),
                         kr(w3[32:, 16 * j:16 * (j + 1)])], axis=0)
        for j in (0, 1)])
    cibd = jnp.concatenate([kr(ci_w[:16]), kr(ci_w[16:32])], axis=0)
    cibd2 = kr(ci_w[32:])
    cpw2bd = kr(cp_w2)
    clsbd = kr(cls_w)

    deg = _deg_pass(ei2, x)
    disP, y1P = _tc1(deg.reshape(2, NP, 128), xv, gmat)
    l1 = _agg1_pass(ei2, y1P.reshape(N_PAD, 16))
    y2s = _tc2(l1.reshape(2, NP, 128), xv, disP, gmat, w1bd, t8(b1),
               t8(bn1_g), t8(bn1_b), t8(bn1_rm), t8(bn1_rv), s32, w2bds)
    l2 = _agg2_pass(ei2, y2s.reshape(NC * N_PAD, 32))
    y3s = _tc3(l2.reshape(2, NP, 256), y2s, disP, s32, pt(b2, 32),
               pt(bn2_g, 32), pt(bn2_b, 32), pt(bn2_rm, 32), pt(bn2_rv, 32),
               w3bds)
    l3 = _agg3_pass(ei2, y3s.reshape(NC * N_PAD, 16))
    outP = _tc4(l3.reshape(2, NP, 128), y3s, disP, xv, gmat, chsel,
                pt(b3, 16), pt(bn3_g, 16), pt(bn3_b, 16), pt(bn3_rm, 16),
                pt(bn3_rv, 16), t8(cp_w1.reshape(-1)), t8(cp_b1), cpw2bd,
                t8(cp_b2), cibd, cibd2, t8(ci_b), clsbd, t8(cls_b))
    return outP.reshape(N_PAD, 1)[:N]


# speedup vs baseline: 52.5478x; 1.0536x over previous
import functools

import jax
import jax.numpy as jnp
from jax import lax
from jax.experimental import pallas as pl
from jax.experimental.pallas import tpu as pltpu
from jax.experimental.pallas import tpu_sc as plsc

N = 50000
N_PAD = 51200
E = 800000
CHUNK = 128
ECH = E // CHUNK
NC = 2
NT = 16
RPT = N // NT
ZROWS = 125


@functools.cache
def _sc_pass(F, gather, split_edges, idx_stride):
    U = 5 if F > 16 else 12
    cpc = ECH // NC if split_edges else ECH
    pairs, leftover = divmod(cpc, 2 * U)
    qp, rp = divmod(pairs, NT)

    def body(ei_ref, tbl_ref, out_ref, ibuf, rows, zbuf, acc, *sems):
        gsem = sems[:U]
        ssem = sems[U:2 * U]
        isem = sems[2 * U:]
        c = lax.axis_index("c")
        s = lax.axis_index("s")
        off = c * idx_stride

        def zrow(i, carry):
            for t in range(F // 16):
                zbuf[i, pl.ds(t * 16, 16)] = jnp.zeros((16,), jnp.float32)
            return carry
        lax.fori_loop(0, ZROWS, zrow, 0)
        if not gather:
            def orow(i, carry):
                for t in range(F // 16):
                    rows[0, i, pl.ds(t * 16, 16)] = jnp.ones((16,), jnp.float32)
                return carry
            lax.fori_loop(0, CHUNK, orow, 0)

        def zcp(k, carry):
            pltpu.sync_copy(zbuf, acc.at[pl.ds(s * RPT + k * ZROWS, ZROWS), :])
            return carry
        lax.fori_loop(0, RPT // ZROWS, zcp, 0)
        plsc.subcore_barrier()

        base_c = c * cpc if split_edges else 0
        pcnt = qp + jnp.where(s < rp, 1, 0)
        pstart = base_c + (s * qp + jnp.minimum(s, rp)) * 2 * U

        def sidx(p, u):
            return ibuf.at[p, u, pl.ds(0, CHUNK)]

        def didx(p, u):
            return ibuf.at[p, u, pl.ds(CHUNK, CHUNK)]

        def scat_desc(p, u):
            return pltpu.make_async_copy(
                rows.at[u], acc.at[didx(p, u)], ssem[u])

        def idx_desc(p):
            return pltpu.make_async_copy(
                ei_ref.at[pl.ds(0, U), :], ibuf.at[p], isem[p])

        def phase(chunk0, p, drain, pre_chunk, first=False):
            if first:
                pltpu.sync_copy(ei_ref.at[pl.ds(chunk0, U), :], ibuf.at[p])
            else:
                idx_desc(p).wait()
            if gather and idx_stride:
                for u in range(U):
                    for t in range(CHUNK // 16):
                        sl = pl.ds(t * 16, 16)
                        ibuf[p, u, sl] = ibuf[p, u, sl] + off
            if drain:
                for u in range(U):
                    scat_desc(1 - p, u).wait()
            if pre_chunk is not None:
                pltpu.async_copy(ei_ref.at[pl.ds(pre_chunk, U), :],
                                 ibuf.at[1 - p], isem[1 - p])
            if gather:
                for u in range(U):
                    pltpu.async_copy(tbl_ref.at[sidx(p, u)], rows.at[u],
                                     gsem[u])
            for u in range(U):
                if gather:
                    pltpu.make_async_copy(tbl_ref.at[sidx(p, u)], rows.at[u],
                                          gsem[u]).wait()
                    src = rows.at[u]
                else:
                    src = rows.at[0]
                pltpu.async_copy(src, acc.at[didx(p, u)], ssem[u], add=True)

        phase(pstart, 0, drain=False, pre_chunk=pstart + U, first=True)
        phase(pstart + U, 1, drain=True,
              pre_chunk=jnp.minimum(pstart + 2 * U, ECH - U))

        def pair_body(k, carry):
            c0 = pstart + k * 2 * U
            phase(c0, 0, drain=True, pre_chunk=c0 + U)
            phase(c0 + U, 1, drain=True,
                  pre_chunk=jnp.minimum(c0 + 2 * U, ECH - U))
            return carry
        lax.fori_loop(1, pcnt, pair_body, 0)

        for u in range(U):
            scat_desc(1, u).wait()
        idx_desc(0).wait()

        if leftover:
            @pl.when(s == NT - 1)
            def _tail():
                for t in range(leftover):
                    ch = base_c + pairs * 2 * U + t
                    pltpu.sync_copy(ei_ref.at[pl.ds(ch, 1), :],
                                    ibuf.at[0, pl.ds(0, 1), :])
                    if gather:
                        if idx_stride:
                            for tt in range(CHUNK // 16):
                                sl = pl.ds(tt * 16, 16)
                                ibuf[0, 0, sl] = ibuf[0, 0, sl] + off
                        pltpu.async_copy(tbl_ref.at[sidx(0, 0)], rows.at[1],
                                         gsem[0]).wait()
                        pltpu.sync_copy(rows.at[1], acc.at[didx(0, 0)],
                                        add=True)
                    else:
                        pltpu.sync_copy(rows.at[0], acc.at[didx(0, 0)],
                                        add=True)

        plsc.subcore_barrier()
        pltpu.sync_copy(acc.at[pl.ds(s * RPT, RPT), :],
                        out_ref.at[pl.ds(c * N_PAD + s * RPT, RPT), :])

    return pl.kernel(
        body,
        out_type=jax.ShapeDtypeStruct((NC * N_PAD, F), jnp.float32),
        mesh=plsc.VectorSubcoreMesh(core_axis_name="c", subcore_axis_name="s",
                                    num_cores=NC, num_subcores=NT),
        compiler_params=pltpu.CompilerParams(use_tc_tiling_on_sc=False),
        scratch_types=[
            pltpu.VMEM((2, U, 2 * CHUNK), jnp.int32),
            pltpu.VMEM((U, CHUNK, F), jnp.float32),
            pltpu.VMEM((ZROWS, F), jnp.float32),
            pltpu.VMEM_SHARED((N, F), jnp.float32),
        ] + [pltpu.SemaphoreType.DMA] * (2 * U + 2),
    )


def _deg_pass(ei, t):
    return _sc_pass(16, gather=False, split_edges=True, idx_stride=0)(ei, t)


def _agg1_pass(ei, t):
    return _sc_pass(16, gather=True, split_edges=True, idx_stride=0)(ei, t)


def _agg2_pass(ei, t):
    return _sc_pass(32, gather=True, split_edges=False, idx_stride=N_PAD)(ei, t)


def _agg3_pass(ei, t):
    return _sc_pass(16, gather=True, split_edges=False, idx_stride=N_PAD)(ei, t)


NP = N_PAD // 8
BP = 1280
NBP = NP // BP


def _bn_st(g, b, rm, rv):
    s = g * lax.rsqrt(rv + 1e-5)
    return s, b - rm * s


def _tc1_body(degr, xv, gmat, dis_o, y1_o):
    disv = lax.rsqrt(degr[0] + degr[1] + 1.0)
    dis_o[...] = disv
    y1_o[...] = disv * jnp.dot(xv[...], gmat[...],
                               preferred_element_type=jnp.float32)


def _tc2_body(l1r, xv, disP, gmat, w1bd, b1t, g, bb, rm, rv, s32, w2bd, y2_o):
    d = disP[...]
    x16 = jnp.dot(xv[...], gmat[...], preferred_element_type=jnp.float32)
    aggx = d * (l1r[0] + l1r[1]) + (d * d) * x16
    z = jnp.dot(aggx, w1bd[...], preferred_element_type=jnp.float32) + b1t[...]
    sc, sh = _bn_st(g[...], bb[...], rm[...], rv[...])
    h1 = jnp.maximum(z * sc + sh, 0.0)
    d32 = jnp.dot(d, s32[...], preferred_element_type=jnp.float32)
    y2_o[0] = d32 * jnp.dot(h1, w2bd[0], preferred_element_type=jnp.float32)


def _tc3_body(l2r, y2s, disP, s32, b2pt, g, bb, rm, rv, w3bd, y3_o):
    d = disP[...]
    d32 = jnp.dot(d, s32[...], preferred_element_type=jnp.float32)
    conv = jnp.concatenate([d32 * (l2r[0] + y2s[0]),
                            d32 * (l2r[1] + y2s[1])], axis=1) + b2pt[...]
    sc, sh = _bn_st(g[...], bb[...], rm[...], rv[...])
    h2 = jnp.maximum(conv * sc + sh, 0.0)
    y3_o[0] = d * jnp.dot(h2, w3bd[0], preferred_element_type=jnp.float32)


def _tc4_body(l3r, y3s, disP, xv, gmat, chsel, b3pt, g, bb, rm, rv,
              cpw1t, cpb1t, cpw2bd, cpb2t, cibd, cibd2, cibt, clsbd, clsbt,
              out_o):
    d = disP[...]
    conv = jnp.concatenate([d * (l3r[0] + y3s[0]),
                            d * (l3r[1] + y3s[1])], axis=1) + b3pt[...]
    sc, sh = _bn_st(g[...], bb[...], rm[...], rv[...])
    h3 = jnp.maximum(conv * sc + sh, 0.0)
    x16 = jnp.dot(xv[...], gmat[...], preferred_element_type=jnp.float32)
    chP = jnp.dot(x16, chsel[...], preferred_element_type=jnp.float32)
    pc1 = jnp.maximum(chP * cpw1t[...] + cpb1t[...], 0.0)
    pc = jnp.dot(pc1, cpw2bd[...],
                 preferred_element_type=jnp.float32) + cpb2t[...]
    hh = (jnp.dot(h3, cibd[...], preferred_element_type=jnp.float32)
          + jnp.dot(pc, cibd2[...], preferred_element_type=jnp.float32)
          + cibt[...])
    hh = jnp.maximum(hh, 0.0)
    logit = jnp.dot(hh, clsbd[...],
                    preferred_element_type=jnp.float32) + clsbt[...]
    out_o[...] = jax.nn.sigmoid(logit)


def _prow(fdim):
    return pl.BlockSpec((BP, fdim), lambda i, *_: (i, 0))


def _pfull(shape):
    return pl.BlockSpec(shape, lambda *_: tuple(0 for _ in shape))


def _p2(fdim):
    return pl.BlockSpec((2, BP, fdim), lambda i, *_: (0, i, 0))


def _build_tc(interpret=False):
    tc1 = pl.pallas_call(
        _tc1_body,
        grid=(NBP,),
        in_specs=[_p2(128), _prow(80), _pfull((80, 128))],
        out_specs=[_prow(128), _prow(128)],
        out_shape=[jax.ShapeDtypeStruct((NP, 128), jnp.float32),
                   jax.ShapeDtypeStruct((NP, 128), jnp.float32)],
        interpret=interpret,
    )
    tc2 = pl.pallas_call(
        _tc2_body,
        grid=(NBP, 2),
        in_specs=[_p2(128), _prow(80), _prow(128), _pfull((80, 128)),
                  _pfull((128, 1024)), _pfull((1, 1024)),
                  _pfull((1, 1024)), _pfull((1, 1024)),
                  _pfull((1, 1024)), _pfull((1, 1024)),
                  _pfull((128, 256)),
                  pl.BlockSpec((1, 1024, 256), lambda i, j: (j, 0, 0))],
        out_specs=pl.BlockSpec((1, BP, 256), lambda i, j: (j, i, 0)),
        out_shape=jax.ShapeDtypeStruct((2, NP, 256), jnp.float32),
        interpret=interpret,
    )
    tc3 = pl.pallas_call(
        _tc3_body,
        grid=(NBP, 2),
        in_specs=[_p2(256), _p2(256), _prow(128), _pfull((128, 256)),
                  _pfull((1, 512)),
                  _pfull((1, 512)), _pfull((1, 512)),
                  _pfull((1, 512)), _pfull((1, 512)),
                  pl.BlockSpec((1, 512, 128), lambda i, j: (j, 0, 0))],
        out_specs=pl.BlockSpec((1, BP, 128), lambda i, j: (j, i, 0)),
        out_shape=jax.ShapeDtypeStruct((2, NP, 128), jnp.float32),
        interpret=interpret,
    )
    tc4 = pl.pallas_call(
        _tc4_body,
        grid=(NBP,),
        in_specs=[_p2(128), _p2(128), _prow(128), _prow(80),
                  _pfull((80, 128)), _pfull((128, 256)), _pfull((1, 256)),
                  _pfull((1, 256)), _pfull((1, 256)),
                  _pfull((1, 256)), _pfull((1, 256)),
                  _pfull((1, 256)), _pfull((1, 256)),
                  _pfull((256, 128)), _pfull((1, 128)),
                  _pfull((256, 256)), _pfull((128, 256)), _pfull((1, 256)),
                  _pfull((256, 8)), _pfull((1, 8))],
        out_specs=_prow(8),
        out_shape=jax.ShapeDtypeStruct((NP, 8), jnp.float32),
        interpret=interpret,
    )
    return tc1, tc2, tc3, tc4


_tc1, _tc2, _tc3, _tc4 = _build_tc()


def kernel(x, edge_index, cp_w1, cp_b1, cp_w2, cp_b2, w1, b1, w2, b2, w3, b3,
           bn1_g, bn1_b, bn1_rm, bn1_rv, bn2_g, bn2_b, bn2_rm, bn2_rv,
           bn3_g, bn3_b, bn3_rm, bn3_rv, ci_w, ci_b, cls_w, cls_b):
    f32 = jnp.float32
    ei2 = (edge_index.reshape(2, ECH, CHUNK)
           .transpose(1, 0, 2).reshape(ECH, 2 * CHUNK))
    xv = jnp.pad(x.reshape(N // 8, 80), ((0, NP - N // 8), (0, 0)))
    ey8 = jnp.eye(8, dtype=f32)
    kr = lambda w: jnp.kron(ey8, w)
    t8 = lambda v: jnp.tile(v.reshape(-1), 8).reshape(1, -1)
    pt = lambda v, h: jnp.concatenate([t8(v[:h]), t8(v[h:])], axis=1)

    gmat = kr(jnp.eye(10, 16, dtype=f32))
    s32 = kr(jnp.zeros((16, 32), f32).at[0, :].set(1.0))
    chsel = kr(jnp.zeros((16, 32), f32).at[8, :].set(1.0))
    w1bd = kr(jnp.pad(w1, ((0, 6), (0, 0))))
    w2bds = jnp.stack([kr(w2[:, :32]), kr(w2[:, 32:])])
    w3bds = jnp.stack([
        jnp.concatenate([kr(w3[:32, 16 * j:16 * (j + 1)]),
                         kr(w3[32:, 16 * j:16 * (j + 1)])], axis=0)
        for j in (0, 1)])
    cibd = jnp.concatenate([kr(ci_w[:16]), kr(ci_w[16:32])], axis=0)
    cibd2 = kr(ci_w[32:])
    cpw2bd = kr(cp_w2)
    clsbd = kr(cls_w)

    deg = _deg_pass(ei2, x)
    disP, y1P = _tc1(deg.reshape(2, NP, 128), xv, gmat)
    l1 = _agg1_pass(ei2, y1P.reshape(N_PAD, 16))
    y2s = _tc2(l1.reshape(2, NP, 128), xv, disP, gmat, w1bd, t8(b1),
               t8(bn1_g), t8(bn1_b), t8(bn1_rm), t8(bn1_rv), s32, w2bds)
    l2 = _agg2_pass(ei2, y2s.reshape(NC * N_PAD, 32))
    y3s = _tc3(l2.reshape(2, NP, 256), y2s, disP, s32, pt(b2, 32),
               pt(bn2_g, 32), pt(bn2_b, 32), pt(bn2_rm, 32), pt(bn2_rv, 32),
               w3bds)
    l3 = _agg3_pass(ei2, y3s.reshape(NC * N_PAD, 16))
    outP = _tc4(l3.reshape(2, NP, 128), y3s, disP, xv, gmat, chsel,
                pt(b3, 16), pt(bn3_g, 16), pt(bn3_b, 16), pt(bn3_rm, 16),
                pt(bn3_rv, 16), t8(cp_w1.reshape(-1)), t8(cp_b1), cpw2bd,
                t8(cp_b2), cibd, cibd2, t8(ci_b), clsbd, t8(cls_b))
    return outP.reshape(N_PAD, 1)[:N]
